# baseline (device time: 281043 ns/iter reference)
import jax
import jax.numpy as jnp
from jax import lax
from jax.experimental import pallas as pl
from jax.experimental.pallas import tpu as pltpu

N_DEV = 4
K_BLK = 1024
N_SPLIT = 8


def kernel(dy, W):
    m, k = dy.shape
    n, k2 = W.shape
    assert k == k2
    nk = k // K_BLK
    chunk = m // N_DEV
    half = n // 2
    nq = n // N_SPLIT

    def body(dy_ref, w_ref, out_ref,
             send_cw, send_ccw, recv_cw, recv_ccw, send_sems, recv_sems):
        kk = pl.program_id(0)

        dy_bf = dy_ref[...].astype(jnp.bfloat16)
        for q in range(N_SPLIT):
            w_q = w_ref[pl.ds(q * nq, nq), :].astype(jnp.bfloat16)
            acc_q = lax.dot_general(
                dy_bf, w_q,
                dimension_numbers=(((1,), (1,)), ((), ())),
                preferred_element_type=jnp.float32,
            )

            @pl.when(kk == 0)
            def _(q=q, acc_q=acc_q):
                out_ref[:, pl.ds(q * nq, nq)] = acc_q

            @pl.when(kk > 0)
            def _(q=q, acc_q=acc_q):
                out_ref[:, pl.ds(q * nq, nq)] += acc_q

        @pl.when(kk == nk - 1)
        def _comm():
            my = lax.axis_index("i")
            left = (my - 1) % N_DEV
            right = (my + 1) % N_DEV

            def rdma_pair(slot, cw_src, ccw_src):
                cw = pltpu.make_async_remote_copy(
                    src_ref=cw_src,
                    dst_ref=recv_cw.at[slot],
                    send_sem=send_sems.at[0],
                    recv_sem=recv_sems.at[0, slot],
                    device_id=(right,),
                    device_id_type=pl.DeviceIdType.MESH,
                )
                ccw = pltpu.make_async_remote_copy(
                    src_ref=ccw_src,
                    dst_ref=recv_ccw.at[slot],
                    send_sem=send_sems.at[1],
                    recv_sem=recv_sems.at[1, slot],
                    device_id=(left,),
                    device_id_type=pl.DeviceIdType.MESH,
                )
                cw.start()
                ccw.start()
                cw.wait()
                ccw.wait()

            barrier = pltpu.get_barrier_semaphore()
            for nbr in (left, right):
                pl.semaphore_signal(
                    barrier, inc=1,
                    device_id=(nbr,), device_id_type=pl.DeviceIdType.MESH,
                )
            pl.semaphore_wait(barrier, 2)

            send_cw[...] = out_ref[
                pl.ds(my * chunk, chunk), pl.ds(0, half)
            ].astype(jnp.bfloat16)
            send_ccw[...] = out_ref[
                pl.ds(my * chunk, chunk), pl.ds(half, half)
            ].astype(jnp.bfloat16)
            for s in range(N_DEV - 1):
                slot = s % 2
                cw_recv = (my - s - 1) % N_DEV
                ccw_recv = (my + s + 1) % N_DEV
                rdma_pair(slot, send_cw, send_ccw)
                sum_cw = (
                    out_ref[pl.ds(cw_recv * chunk, chunk), pl.ds(0, half)]
                    + recv_cw[slot].astype(jnp.float32)
                )
                sum_ccw = (
                    out_ref[pl.ds(ccw_recv * chunk, chunk), pl.ds(half, half)]
                    + recv_ccw[slot].astype(jnp.float32)
                )
                out_ref[pl.ds(cw_recv * chunk, chunk), pl.ds(0, half)] = sum_cw
                out_ref[pl.ds(ccw_recv * chunk, chunk), pl.ds(half, half)] = sum_ccw
                if s < N_DEV - 2:
                    send_cw[...] = sum_cw.astype(jnp.bfloat16)
                    send_ccw[...] = sum_ccw.astype(jnp.bfloat16)

            send_cw[...] = out_ref[
                pl.ds(((my + 1) % N_DEV) * chunk, chunk), pl.ds(0, half)
            ].astype(jnp.bfloat16)
            send_ccw[...] = out_ref[
                pl.ds(((my - 1) % N_DEV) * chunk, chunk), pl.ds(half, half)
            ].astype(jnp.bfloat16)
            for s in range(N_DEV - 1):
                slot = (N_DEV - 1 + s) % 2
                cw_recv = (my - s) % N_DEV
                ccw_recv = (my + s) % N_DEV
                if s == 0:
                    rdma_pair(slot, send_cw, send_ccw)
                else:
                    rdma_pair(slot, recv_cw.at[1 - slot], recv_ccw.at[1 - slot])
                out_ref[pl.ds(cw_recv * chunk, chunk), pl.ds(0, half)] = (
                    recv_cw[slot].astype(jnp.float32)
                )
                out_ref[pl.ds(ccw_recv * chunk, chunk), pl.ds(half, half)] = (
                    recv_ccw[slot].astype(jnp.float32)
                )

    return pl.pallas_call(
        body,
        grid=(nk,),
        in_specs=[
            pl.BlockSpec((m, K_BLK), lambda kk: (0, kk)),
            pl.BlockSpec((n, K_BLK), lambda kk: (0, kk)),
        ],
        out_specs=pl.BlockSpec((m, n), lambda kk: (0, 0)),
        out_shape=jax.ShapeDtypeStruct((m, n), jnp.float32),
        scratch_shapes=[
            pltpu.VMEM((chunk, half), jnp.bfloat16),
            pltpu.VMEM((chunk, half), jnp.bfloat16),
            pltpu.VMEM((2, chunk, half), jnp.bfloat16),
            pltpu.VMEM((2, chunk, half), jnp.bfloat16),
            pltpu.SemaphoreType.DMA((2,)),
            pltpu.SemaphoreType.DMA((2, 2)),
        ],
        compiler_params=pltpu.CompilerParams(
            collective_id=0,
            vmem_limit_bytes=int(63.9 * 1024 * 1024),
        ),
    )(dy, W)


# device time: 207450 ns/iter; 1.3548x vs baseline; 1.3548x over previous
import jax
import jax.numpy as jnp
from jax import lax
from jax.experimental import pallas as pl
from jax.experimental.pallas import tpu as pltpu

N_DEV = 4
K_BLK = 512


def kernel(dy, W):
    m, k = dy.shape
    n, k2 = W.shape
    assert k == k2
    nk = k // K_BLK
    chunk = m // N_DEV
    half = n // 2

    def body(dy_ref, w_ref, out_ref,
             send_cw, send_ccw, recv_cw, recv_ccw, send_sems, recv_sems):
        kk = pl.program_id(0)
        my = lax.axis_index("i")
        left = (my - 1) % N_DEV
        right = (my + 1) % N_DEV
        barrier = pltpu.get_barrier_semaphore()

        @pl.when(kk == 0)
        def _():
            for nbr in (left, right):
                pl.semaphore_signal(
                    barrier, inc=1,
                    device_id=(nbr,), device_id_type=pl.DeviceIdType.MESH,
                )

        acc = lax.dot_general(
            dy_ref[...].astype(jnp.bfloat16),
            w_ref[...].astype(jnp.bfloat16),
            dimension_numbers=(((1,), (1,)), ((), ())),
            preferred_element_type=jnp.float32,
        )

        @pl.when(kk == 0)
        def _():
            out_ref[...] = acc

        @pl.when(kk > 0)
        def _():
            out_ref[...] += acc

        @pl.when(kk == nk - 1)
        def _comm():

            def rdma_pair(slot, cw_src, ccw_src):
                cw = pltpu.make_async_remote_copy(
                    src_ref=cw_src,
                    dst_ref=recv_cw.at[slot],
                    send_sem=send_sems.at[0],
                    recv_sem=recv_sems.at[0, slot],
                    device_id=(right,),
                    device_id_type=pl.DeviceIdType.MESH,
                )
                ccw = pltpu.make_async_remote_copy(
                    src_ref=ccw_src,
                    dst_ref=recv_ccw.at[slot],
                    send_sem=send_sems.at[1],
                    recv_sem=recv_sems.at[1, slot],
                    device_id=(left,),
                    device_id_type=pl.DeviceIdType.MESH,
                )
                cw.start()
                ccw.start()
                cw.wait()
                ccw.wait()

            pl.semaphore_wait(barrier, 2)

            send_cw[...] = out_ref[
                pl.ds(my * chunk, chunk), pl.ds(0, half)
            ].astype(jnp.bfloat16)
            send_ccw[...] = out_ref[
                pl.ds(my * chunk, chunk), pl.ds(half, half)
            ].astype(jnp.bfloat16)
            for s in range(N_DEV - 1):
                slot = s % 2
                cw_recv = (my - s - 1) % N_DEV
                ccw_recv = (my + s + 1) % N_DEV
                rdma_pair(slot, send_cw, send_ccw)
                sum_cw = (
                    out_ref[pl.ds(cw_recv * chunk, chunk), pl.ds(0, half)]
                    + recv_cw[slot].astype(jnp.float32)
                )
                sum_ccw = (
                    out_ref[pl.ds(ccw_recv * chunk, chunk), pl.ds(half, half)]
                    + recv_ccw[slot].astype(jnp.float32)
                )
                out_ref[pl.ds(cw_recv * chunk, chunk), pl.ds(0, half)] = sum_cw
                out_ref[pl.ds(ccw_recv * chunk, chunk), pl.ds(half, half)] = sum_ccw
                if s < N_DEV - 2:
                    send_cw[...] = sum_cw.astype(jnp.bfloat16)
                    send_ccw[...] = sum_ccw.astype(jnp.bfloat16)

            send_cw[...] = out_ref[
                pl.ds(((my + 1) % N_DEV) * chunk, chunk), pl.ds(0, half)
            ].astype(jnp.bfloat16)
            send_ccw[...] = out_ref[
                pl.ds(((my - 1) % N_DEV) * chunk, chunk), pl.ds(half, half)
            ].astype(jnp.bfloat16)
            for s in range(N_DEV - 1):
                slot = (N_DEV - 1 + s) % 2
                cw_recv = (my - s) % N_DEV
                ccw_recv = (my + s) % N_DEV
                if s == 0:
                    rdma_pair(slot, send_cw, send_ccw)
                else:
                    rdma_pair(slot, recv_cw.at[1 - slot], recv_ccw.at[1 - slot])
                out_ref[pl.ds(cw_recv * chunk, chunk), pl.ds(0, half)] = (
                    recv_cw[slot].astype(jnp.float32)
                )
                out_ref[pl.ds(ccw_recv * chunk, chunk), pl.ds(half, half)] = (
                    recv_ccw[slot].astype(jnp.float32)
                )

    return pl.pallas_call(
        body,
        grid=(nk,),
        in_specs=[
            pl.BlockSpec((m, K_BLK), lambda kk: (0, kk)),
            pl.BlockSpec((n, K_BLK), lambda kk: (0, kk)),
        ],
        out_specs=pl.BlockSpec((m, n), lambda kk: (0, 0)),
        out_shape=jax.ShapeDtypeStruct((m, n), jnp.float32),
        scratch_shapes=[
            pltpu.VMEM((chunk, half), jnp.bfloat16),
            pltpu.VMEM((chunk, half), jnp.bfloat16),
            pltpu.VMEM((2, chunk, half), jnp.bfloat16),
            pltpu.VMEM((2, chunk, half), jnp.bfloat16),
            pltpu.SemaphoreType.DMA((2,)),
            pltpu.SemaphoreType.DMA((2, 2)),
        ],
        compiler_params=pltpu.CompilerParams(
            collective_id=0,
            vmem_limit_bytes=int(63.9 * 1024 * 1024),
        ),
    )(dy, W)
